# baseline (speedup 1.0000x reference)
import functools

import jax
import jax.numpy as jnp
from jax.experimental import pallas as pl
from jax.experimental.pallas import tpu as pltpu
from jax.experimental.pallas import tpu_sc as plsc

_B, _S, _D = 4096, 50, 64
_N = _B * _S
_WINDOW = 400


def kernel(x, emb_weight):
    idx = x.reshape(_N).astype(jnp.int32)
    table = jnp.pad(emb_weight, ((0, 0), (0, 128 - _D)))

    @functools.partial(
        pl.kernel,
        out_type=jax.ShapeDtypeStruct((_N, 128), emb_weight.dtype),
        mesh=plsc.VectorSubcoreMesh(core_axis_name="c", subcore_axis_name="s"),
        compiler_params=pltpu.CompilerParams(use_tc_tiling_on_sc=False),
    )
    def gather_kernel(table_hbm, idx_hbm, out_hbm):
        def body(idx_vmem, out_vmem):
            pltpu.sync_copy(table_hbm.at[idx_vmem], out_vmem)

        pltpu.emit_pipeline(
            body,
            grid=(_N // _WINDOW,),
            in_specs=[pl.BlockSpec((_WINDOW,), index_map=lambda i: (i,))],
            out_specs=[pl.BlockSpec((_WINDOW, 128), index_map=lambda i: (i, 0))],
            core_axis_name=("c", "s"),
            dimension_semantics=(pltpu.PARALLEL,),
        )(idx_hbm, out_hbm)

    out = gather_kernel(table, idx)
    return out[:, :_D].reshape(_B, _S, _D)


# speedup vs baseline: 1.0010x; 1.0010x over previous
import functools

import jax
import jax.numpy as jnp
from jax.experimental import pallas as pl
from jax.experimental.pallas import tpu as pltpu
from jax.experimental.pallas import tpu_sc as plsc

_B, _S, _D = 4096, 50, 64
_N = _B * _S
_WINDOW = 400


def kernel(x, emb_weight):
    idx = x.reshape(_N).astype(jnp.int32)
    table = jnp.pad(emb_weight, ((0, 0), (0, 128 - _D)))

    @functools.partial(
        pl.kernel,
        out_type=jax.ShapeDtypeStruct((_N, 128), emb_weight.dtype),
        mesh=plsc.VectorSubcoreMesh(core_axis_name="c", subcore_axis_name="s"),
    )
    def gather_kernel(table_hbm, idx_hbm, out_hbm):
        def body(idx_vmem, out_vmem):
            pltpu.sync_copy(table_hbm.at[idx_vmem], out_vmem)

        pltpu.emit_pipeline(
            body,
            grid=(_N // _WINDOW,),
            in_specs=[pl.BlockSpec((_WINDOW,), index_map=lambda i: (i,))],
            out_specs=[pl.BlockSpec((_WINDOW, 128), index_map=lambda i: (i, 0))],
            core_axis_name=("c", "s"),
            dimension_semantics=(pltpu.PARALLEL,),
        )(idx_hbm, out_hbm)

    out = gather_kernel(table, idx)
    return out[:, :_D].reshape(_B, _S, _D)
